# baseline (speedup 1.0000x reference)
import functools
import math

import jax
import jax.numpy as jnp
from jax import lax
from jax.experimental import pallas as pl
from jax.experimental.pallas import tpu as pltpu
from jax.experimental.pallas import tpu_sc as plsc

_B, _L, _C, _V = 4096, 200, 128, 1000
_LC = 8


def _sc_call(pT, aT, fT, ptab, ftab):
    info = plsc.get_sparse_core_info()
    nw = info.num_cores * info.num_subcores
    bpw = _B // nw
    mesh = plsc.VectorSubcoreMesh(core_axis_name="c", subcore_axis_name="s")

    @functools.partial(
        pl.kernel,
        out_type=jax.ShapeDtypeStruct((_B, _L * 3 * _C), jnp.float32),
        mesh=mesh,
        scratch_types=[
            pltpu.VMEM((_LC, bpw), jnp.int32),
            pltpu.VMEM((_LC, bpw), jnp.int32),
            pltpu.VMEM((_LC, bpw), jnp.float32),
            pltpu.VMEM((bpw, _C), jnp.float32),
            pltpu.VMEM((bpw, _C), jnp.float32),
            pltpu.VMEM((bpw, _C), jnp.float32),
            pltpu.SemaphoreType.DMA,
        ],
    )
    def k(pT_hbm, aT_hbm, fT_hbm, ptab_hbm, ftab_hbm, out_hbm,
          pidx, fidx, a1c, prows, frows, a1exp, sem):
        wid = lax.axis_index("s") * info.num_cores + lax.axis_index("c")
        b0 = wid * bpw

        def chunk(lc, carry):
            l0 = lc * _LC
            pltpu.sync_copy(pT_hbm.at[pl.ds(l0, _LC), pl.ds(b0, bpw)], pidx)
            pltpu.sync_copy(fT_hbm.at[pl.ds(l0, _LC), pl.ds(b0, bpw)], fidx)
            pltpu.sync_copy(aT_hbm.at[pl.ds(l0, _LC), pl.ds(b0, bpw)], a1c)
            for li in range(_LC):
                pltpu.async_copy(ptab_hbm.at[pidx.at[li]], prows, sem).wait()
                pltpu.async_copy(ftab_hbm.at[fidx.at[li]], frows, sem).wait()

                def expand(g, c2):
                    v = a1c[li, pl.ds(g * 16, 16)]
                    for k16 in range(16):
                        row = jnp.broadcast_to(v[k16], (16,))
                        for q in range(_C // 16):
                            a1exp[g * 16 + k16, pl.ds(q * 16, 16)] = row
                    return c2
                lax.fori_loop(0, bpw // 16, expand, 0)

                col = (l0 + li) * 3 * _C
                pltpu.sync_copy(
                    prows, out_hbm.at[pl.ds(b0, bpw), pl.ds(col, _C)])
                pltpu.sync_copy(
                    frows, out_hbm.at[pl.ds(b0, bpw), pl.ds(col + _C, _C)])
                pltpu.sync_copy(
                    a1exp,
                    out_hbm.at[pl.ds(b0, bpw), pl.ds(col + 2 * _C, _C)])
            return carry

        lax.fori_loop(0, _L // _LC, chunk, 0)

    return k(pT, aT, fT, ptab, ftab)


@jax.jit
def kernel(phoneme, a1, f2, phoneme_table, f2_table):
    B, L = phoneme.shape
    V, C = phoneme_table.shape
    scale = math.sqrt(C)
    out2 = _sc_call(phoneme.T, a1.T, f2.T,
                    phoneme_table * scale, f2_table * scale)
    out = out2.reshape(B, L, 3 * C)
    return jnp.swapaxes(out, -1, -2)


# speedup vs baseline: 2.4910x; 2.4910x over previous
import math

import jax
import jax.numpy as jnp
from jax.experimental import pallas as pl
from jax.experimental.pallas import tpu as pltpu

_VPAD = 1024
_BB = 8
_BI = 128


def _body(p_ref, a1_ref, f_ref, pt_ref, ft_ref, out_ref):
    L = p_ref.shape[0]
    C = pt_ref.shape[1] // 2
    NL = _BB * L
    j = pl.program_id(1)
    vocab_iota = jax.lax.broadcasted_iota(jnp.int16, (L, _VPAD), 1)

    def emb(idx_ref, tbl):
        idx_lb = pltpu.roll(idx_ref[...], -j * _BB, 1)[:, :_BB]
        idx16 = idx_lb.astype(jnp.int16)
        onehot = jnp.concatenate(
            [jnp.where(vocab_iota == idx16[:, k:k + 1],
                       jnp.bfloat16(1), jnp.bfloat16(0))
             for k in range(_BB)], axis=0).astype(jnp.float8_e4m3fn)
        r = jax.lax.dot_general(onehot, tbl[...], (((1,), (0,)), ((), ())),
                                preferred_element_type=jnp.float32)
        return (r[:, :C] + r[:, C:] * (1.0 / _LO_SHIFT)).reshape(_BB, L, C)

    out_ref[:, :, 0:C] = emb(p_ref, pt_ref)
    out_ref[:, :, C:2 * C] = emb(f_ref, ft_ref)
    a1_lb = pltpu.roll(a1_ref[...], -j * _BB, 1)[:, :_BB]
    for k in range(_BB):
        out_ref[k, :, 2 * C:3 * C] = jnp.broadcast_to(a1_lb[:, k:k + 1], (L, C))


_LO_SHIFT = 64.0


def _split_hi_lo(table):
    hi = jax.lax.optimization_barrier(table.astype(jnp.float8_e4m3fn))
    lo = ((table - hi.astype(jnp.float32)) * _LO_SHIFT).astype(
        jnp.float8_e4m3fn)
    return jnp.concatenate([hi, lo], axis=1)


@jax.jit
def kernel(phoneme, a1, f2, phoneme_table, f2_table):
    B, L = phoneme.shape
    V, C = phoneme_table.shape
    scale = math.sqrt(C)

    def prep(tbl):
        t = jnp.pad(tbl * scale, ((0, _VPAD - V), (0, 0)))
        return _split_hi_lo(t)

    pt = prep(phoneme_table)
    ft = prep(f2_table)

    pT = phoneme.T
    aT = a1.T
    fT = f2.T

    grid = (B // _BI, _BI // _BB)
    out = pl.pallas_call(
        _body,
        grid=grid,
        in_specs=[
            pl.BlockSpec((L, _BI), lambda i, j: (0, i)),
            pl.BlockSpec((L, _BI), lambda i, j: (0, i)),
            pl.BlockSpec((L, _BI), lambda i, j: (0, i)),
            pl.BlockSpec((_VPAD, 2 * C), lambda i, j: (0, 0)),
            pl.BlockSpec((_VPAD, 2 * C), lambda i, j: (0, 0)),
        ],
        out_specs=pl.BlockSpec(
            (_BB, L, 3 * C),
            lambda i, j: (i * (_BI // _BB) + j, 0, 0)),
        out_shape=jax.ShapeDtypeStruct((B, L, 3 * C), jnp.float32),
        compiler_params=pltpu.CompilerParams(
            dimension_semantics=("arbitrary", "arbitrary"),
        ),
    )(pT, aT, fT, pt, ft)
    return jnp.swapaxes(out, -1, -2)
